# Initial kernel scaffold; baseline (speedup 1.0000x reference)
import jax
import jax.numpy as jnp
from jax.experimental import pallas as pl


def kernel(x, y):
    raise NotImplementedError("write your pallas kernel here")


# speedup vs baseline: 20.9936x; 20.9936x over previous
import jax
import jax.numpy as jnp
from jax.experimental import pallas as pl
from jax.experimental.pallas import tpu as pltpu

BINS_ = 10
ROWS = 4096
COLS = 4096
BLK = 256
GRID = ROWS // BLK
TOTAL = float(ROWS * COLS)


def _body(x_ref, y_ref, out_ref, acc_ref):
    pid = pl.program_id(0)

    @pl.when(pid == 0)
    def _init():
        for i in range(2):
            for b in range(BINS_):
                acc_ref[i, b] = 0.0

    x = x_ref[...]
    y = y_ref[...]

    ax = jnp.abs(x)
    e = jnp.exp(-ax)
    s0 = 1.0 / (1.0 + e)
    sig = jnp.where(x >= 0.0, s0, 1.0 - s0)
    g = jnp.abs(sig - y)
    z = g * float(BINS_)
    pe = jnp.maximum(x, 0.0) - x * y + jnp.log1p(e)

    sge = [jnp.sum(pe)]
    cge = []
    for b in range(1, BINS_):
        m = z >= float(b)
        sge.append(jnp.sum(jnp.where(m, pe, 0.0)))
        cge.append(jnp.sum(jnp.where(m, 1.0, 0.0)))

    for b in range(BINS_):
        acc_ref[0, b] += sge[b]
    for b in range(BINS_ - 1):
        acc_ref[1, b] += cge[b]

    @pl.when(pid == GRID - 1)
    def _finish():
        total = 0.0
        n = 0.0
        for b in range(BINS_):
            s_hi = acc_ref[0, b + 1] if b + 1 < BINS_ else 0.0
            c_lo = acc_ref[1, b - 1] if b >= 1 else TOTAL
            c_hi = acc_ref[1, b] if b < BINS_ - 1 else 0.0
            s_b = acc_ref[0, b] - s_hi
            c_b = c_lo - c_hi
            nonempty = c_b > 0.0
            total += jnp.where(nonempty, s_b / jnp.where(nonempty, c_b, 1.0), 0.0)
            n += jnp.where(nonempty, 1.0, 0.0)
        out_ref[0] = total / n


def kernel(x, y):
    out = pl.pallas_call(
        _body,
        grid=(GRID,),
        in_specs=[
            pl.BlockSpec((BLK, COLS), lambda i: (i, 0)),
            pl.BlockSpec((BLK, COLS), lambda i: (i, 0)),
        ],
        out_specs=pl.BlockSpec(memory_space=pltpu.SMEM),
        out_shape=jax.ShapeDtypeStruct((1,), jnp.float32),
        scratch_shapes=[pltpu.SMEM((2, BINS_), jnp.float32)],
    )(x, y)
    return out[0]


# speedup vs baseline: 27.3206x; 1.3014x over previous
import jax
import jax.numpy as jnp
from jax.experimental import pallas as pl
from jax.experimental.pallas import tpu as pltpu

BINS_ = 10
ROWS = 4096
COLS = 4096
BLK = 32
GRID = ROWS // BLK
TOTAL = float(ROWS * COLS)
NACC = 2 * BINS_ - 1


def _body(x_ref, y_ref, out_ref, acc_ref):
    pid = pl.program_id(0)

    @pl.when(pid == 0)
    def _init():
        acc_ref[...] = jnp.zeros_like(acc_ref)

    zero = jnp.zeros((8, 128), jnp.float32)
    sge = [zero] * BINS_
    cge = [zero] * (BINS_ - 1)
    for r in range(BLK // 8):
        for c in range(COLS // 128):
            x = x_ref[r * 8:(r + 1) * 8, c * 128:(c + 1) * 128]
            y = y_ref[r * 8:(r + 1) * 8, c * 128:(c + 1) * 128]
            e = jnp.exp(-jnp.abs(x))
            s0 = 1.0 / (1.0 + e)
            sig = jnp.where(x >= 0.0, s0, 1.0 - s0)
            z = jnp.abs(sig - y) * float(BINS_)
            pe = jnp.maximum(x, 0.0) - x * y + jnp.log1p(e)
            sge[0] = sge[0] + pe
            for b in range(1, BINS_):
                m = z >= float(b)
                sge[b] = sge[b] + jnp.where(m, pe, 0.0)
                cge[b - 1] = cge[b - 1] + jnp.where(m, 1.0, 0.0)

    for b in range(BINS_):
        acc_ref[b] += sge[b]
    for b in range(BINS_ - 1):
        acc_ref[BINS_ + b] += cge[b]

    @pl.when(pid == GRID - 1)
    def _finish():
        s = [jnp.sum(acc_ref[b]) for b in range(BINS_)]
        cg = [jnp.sum(acc_ref[BINS_ + b]) for b in range(BINS_ - 1)]
        total = 0.0
        n = 0.0
        for b in range(BINS_):
            s_hi = s[b + 1] if b + 1 < BINS_ else 0.0
            c_lo = cg[b - 1] if b >= 1 else TOTAL
            c_hi = cg[b] if b < BINS_ - 1 else 0.0
            s_b = s[b] - s_hi
            c_b = c_lo - c_hi
            nonempty = c_b > 0.0
            total += jnp.where(nonempty, s_b / jnp.where(nonempty, c_b, 1.0), 0.0)
            n += jnp.where(nonempty, 1.0, 0.0)
        out_ref[0] = total / n


def kernel(x, y):
    out = pl.pallas_call(
        _body,
        grid=(GRID,),
        in_specs=[
            pl.BlockSpec((BLK, COLS), lambda i: (i, 0)),
            pl.BlockSpec((BLK, COLS), lambda i: (i, 0)),
        ],
        out_specs=pl.BlockSpec(memory_space=pltpu.SMEM),
        out_shape=jax.ShapeDtypeStruct((1,), jnp.float32),
        scratch_shapes=[pltpu.VMEM((NACC, 8, 128), jnp.float32)],
    )(x, y)
    return out[0]


# speedup vs baseline: 27.9276x; 1.0222x over previous
import jax
import jax.numpy as jnp
from jax.experimental import pallas as pl
from jax.experimental.pallas import tpu as pltpu

BINS_ = 10
ROWS = 4096
COLS = 4096
BLK = 32
GRID = ROWS // BLK
TOTAL = float(ROWS * COLS)
NACC = 2 * BINS_ - 1


def _body(x_ref, y_ref, out_ref, acc_ref):
    pid = pl.program_id(0)

    @pl.when(pid == 0)
    def _init():
        acc_ref[...] = jnp.zeros_like(acc_ref)

    zero = jnp.zeros((8, 128), jnp.float32)
    sge = [zero] * BINS_
    cge = [zero] * (BINS_ - 1)
    for r in range(BLK // 8):
        for c in range(COLS // 128):
            x = x_ref[r * 8:(r + 1) * 8, c * 128:(c + 1) * 128]
            y = y_ref[r * 8:(r + 1) * 8, c * 128:(c + 1) * 128]
            e = jnp.exp(-jnp.abs(x))
            s0 = 1.0 / (1.0 + e)
            sig = jnp.where(x >= 0.0, s0, 1.0 - s0)
            z = jnp.abs(sig - y) * float(BINS_)
            pe = jnp.maximum(x, 0.0) - x * y + jnp.log1p(e)
            sge[0] = sge[0] + pe
            for b in range(1, BINS_):
                mf = jnp.where(z >= float(b), 1.0, 0.0)
                sge[b] = sge[b] + pe * mf
                cge[b - 1] = cge[b - 1] + mf

    for b in range(BINS_):
        acc_ref[b] += sge[b]
    for b in range(BINS_ - 1):
        acc_ref[BINS_ + b] += cge[b]

    @pl.when(pid == GRID - 1)
    def _finish():
        s = [jnp.sum(acc_ref[b]) for b in range(BINS_)]
        cg = [jnp.sum(acc_ref[BINS_ + b]) for b in range(BINS_ - 1)]
        total = 0.0
        n = 0.0
        for b in range(BINS_):
            s_hi = s[b + 1] if b + 1 < BINS_ else 0.0
            c_lo = cg[b - 1] if b >= 1 else TOTAL
            c_hi = cg[b] if b < BINS_ - 1 else 0.0
            s_b = s[b] - s_hi
            c_b = c_lo - c_hi
            nonempty = c_b > 0.0
            total += jnp.where(nonempty, s_b / jnp.where(nonempty, c_b, 1.0), 0.0)
            n += jnp.where(nonempty, 1.0, 0.0)
        out_ref[0] = total / n


def kernel(x, y):
    out = pl.pallas_call(
        _body,
        grid=(GRID,),
        in_specs=[
            pl.BlockSpec((BLK, COLS), lambda i: (i, 0)),
            pl.BlockSpec((BLK, COLS), lambda i: (i, 0)),
        ],
        out_specs=pl.BlockSpec(memory_space=pltpu.SMEM),
        out_shape=jax.ShapeDtypeStruct((1,), jnp.float32),
        scratch_shapes=[pltpu.VMEM((NACC, 8, 128), jnp.float32)],
    )(x, y)
    return out[0]


# speedup vs baseline: 30.1543x; 1.0797x over previous
import jax
import jax.numpy as jnp
from jax.experimental import pallas as pl
from jax.experimental.pallas import tpu as pltpu

BINS_ = 10
ROWS = 4096
COLS = 4096
BLK = 32
GRID = ROWS // BLK
TOTAL = float(ROWS * COLS)
NACC = 2 * BINS_ - 1


def _body(x_ref, y_ref, out_ref, acc_ref):
    pid = pl.program_id(0)

    @pl.when(pid == 0)
    def _init():
        acc_ref[...] = jnp.zeros_like(acc_ref)

    zero = jnp.zeros((8, 128), jnp.float32)
    for r in range(BLK // 8):
        sge = [zero] * BINS_
        cge = [zero] * (BINS_ - 1)
        for c in range(COLS // 128):
            x = x_ref[r * 8:(r + 1) * 8, c * 128:(c + 1) * 128]
            y = y_ref[r * 8:(r + 1) * 8, c * 128:(c + 1) * 128]
            e = jnp.exp(-jnp.abs(x))
            s0 = 1.0 / (1.0 + e)
            sig = jnp.where(x >= 0.0, s0, 1.0 - s0)
            z = jnp.abs(sig - y) * float(BINS_)
            pe = jnp.maximum(x, 0.0) - x * y + jnp.log1p(e)
            sge[0] = sge[0] + pe
            for b in range(1, BINS_):
                mf = jnp.where(z >= float(b), 1.0, 0.0)
                sge[b] = sge[b] + pe * mf
                cge[b - 1] = cge[b - 1] + mf

        for b in range(BINS_):
            acc_ref[b] += sge[b]
        for b in range(BINS_ - 1):
            acc_ref[BINS_ + b] += cge[b]

    @pl.when(pid == GRID - 1)
    def _finish():
        s = [jnp.sum(acc_ref[b]) for b in range(BINS_)]
        cg = [jnp.sum(acc_ref[BINS_ + b]) for b in range(BINS_ - 1)]
        total = 0.0
        n = 0.0
        for b in range(BINS_):
            s_hi = s[b + 1] if b + 1 < BINS_ else 0.0
            c_lo = cg[b - 1] if b >= 1 else TOTAL
            c_hi = cg[b] if b < BINS_ - 1 else 0.0
            s_b = s[b] - s_hi
            c_b = c_lo - c_hi
            nonempty = c_b > 0.0
            total += jnp.where(nonempty, s_b / jnp.where(nonempty, c_b, 1.0), 0.0)
            n += jnp.where(nonempty, 1.0, 0.0)
        out_ref[0] = total / n


def kernel(x, y):
    out = pl.pallas_call(
        _body,
        grid=(GRID,),
        in_specs=[
            pl.BlockSpec((BLK, COLS), lambda i: (i, 0)),
            pl.BlockSpec((BLK, COLS), lambda i: (i, 0)),
        ],
        out_specs=pl.BlockSpec(memory_space=pltpu.SMEM),
        out_shape=jax.ShapeDtypeStruct((1,), jnp.float32),
        scratch_shapes=[pltpu.VMEM((NACC, 8, 128), jnp.float32)],
    )(x, y)
    return out[0]


# speedup vs baseline: 31.8936x; 1.0577x over previous
import jax
import jax.numpy as jnp
from jax.experimental import pallas as pl
from jax.experimental.pallas import tpu as pltpu

BINS_ = 10
ROWS = 4096
COLS = 4096
BLK = 32
GRID = ROWS // BLK
TOTAL = float(ROWS * COLS)
NACC = 2 * BINS_ - 1


def _body(x_ref, y_ref, out_ref, acc_ref):
    pid = pl.program_id(0)

    @pl.when(pid == 0)
    def _init():
        acc_ref[...] = jnp.zeros_like(acc_ref)

    zero = jnp.zeros((8, 128), jnp.float32)
    for r in range(BLK // 8):
        sge = [zero] * BINS_
        cge = [zero] * (BINS_ - 1)
        for c in range(COLS // 128):
            x = x_ref[r * 8:(r + 1) * 8, c * 128:(c + 1) * 128]
            y = y_ref[r * 8:(r + 1) * 8, c * 128:(c + 1) * 128]
            e = jnp.exp(-jnp.abs(x))
            u = 1.0 + e
            s0 = 1.0 / u
            sig = jnp.where(x >= 0.0, s0, 1.0 - s0)
            g = jnp.abs(sig - y)
            pe = jnp.maximum(x, 0.0) - x * y + jnp.log(u)
            sge[0] = sge[0] + pe
            for b in range(1, BINS_):
                mf = jnp.where(g >= float(b) / float(BINS_), 1.0, 0.0)
                sge[b] = sge[b] + pe * mf
                cge[b - 1] = cge[b - 1] + mf

        for b in range(BINS_):
            acc_ref[b] += sge[b]
        for b in range(BINS_ - 1):
            acc_ref[BINS_ + b] += cge[b]

    @pl.when(pid == GRID - 1)
    def _finish():
        s = [jnp.sum(acc_ref[b]) for b in range(BINS_)]
        cg = [jnp.sum(acc_ref[BINS_ + b]) for b in range(BINS_ - 1)]
        total = 0.0
        n = 0.0
        for b in range(BINS_):
            s_hi = s[b + 1] if b + 1 < BINS_ else 0.0
            c_lo = cg[b - 1] if b >= 1 else TOTAL
            c_hi = cg[b] if b < BINS_ - 1 else 0.0
            s_b = s[b] - s_hi
            c_b = c_lo - c_hi
            nonempty = c_b > 0.0
            total += jnp.where(nonempty, s_b / jnp.where(nonempty, c_b, 1.0), 0.0)
            n += jnp.where(nonempty, 1.0, 0.0)
        out_ref[0] = total / n


def kernel(x, y):
    out = pl.pallas_call(
        _body,
        grid=(GRID,),
        in_specs=[
            pl.BlockSpec((BLK, COLS), lambda i: (i, 0)),
            pl.BlockSpec((BLK, COLS), lambda i: (i, 0)),
        ],
        out_specs=pl.BlockSpec(memory_space=pltpu.SMEM),
        out_shape=jax.ShapeDtypeStruct((1,), jnp.float32),
        scratch_shapes=[pltpu.VMEM((NACC, 8, 128), jnp.float32)],
    )(x, y)
    return out[0]


# speedup vs baseline: 33.9748x; 1.0653x over previous
import jax
import jax.numpy as jnp
from jax.experimental import pallas as pl
from jax.experimental.pallas import tpu as pltpu

BINS_ = 10
ROWS = 4096
COLS = 4096
BLK = 64
GRID = ROWS // BLK
TOTAL = float(ROWS * COLS)
NACC = 2 * BINS_ - 1


def _body(x_ref, y_ref, out_ref, acc_ref):
    pid = pl.program_id(0)

    @pl.when(pid == 0)
    def _init():
        acc_ref[...] = jnp.zeros_like(acc_ref)

    zero = jnp.zeros((8, 128), jnp.float32)
    for r in range(BLK // 8):
        sge = [zero] * BINS_
        cge = [zero] * (BINS_ - 1)
        for c in range(COLS // 128):
            x = x_ref[r * 8:(r + 1) * 8, c * 128:(c + 1) * 128]
            y = y_ref[r * 8:(r + 1) * 8, c * 128:(c + 1) * 128]
            e = jnp.exp(-jnp.abs(x))
            u = 1.0 + e
            s0 = 1.0 / u
            sig = jnp.where(x >= 0.0, s0, 1.0 - s0)
            g = jnp.abs(sig - y)
            pe = jnp.maximum(x, 0.0) - x * y + jnp.log(u)
            sge[0] = sge[0] + pe
            for b in range(1, BINS_):
                mf = jnp.where(g >= float(b) / float(BINS_), 1.0, 0.0)
                sge[b] = sge[b] + pe * mf
                cge[b - 1] = cge[b - 1] + mf

        for b in range(BINS_):
            acc_ref[b] += sge[b]
        for b in range(BINS_ - 1):
            acc_ref[BINS_ + b] += cge[b]

    @pl.when(pid == GRID - 1)
    def _finish():
        s = [jnp.sum(acc_ref[b]) for b in range(BINS_)]
        cg = [jnp.sum(acc_ref[BINS_ + b]) for b in range(BINS_ - 1)]
        total = 0.0
        n = 0.0
        for b in range(BINS_):
            s_hi = s[b + 1] if b + 1 < BINS_ else 0.0
            c_lo = cg[b - 1] if b >= 1 else TOTAL
            c_hi = cg[b] if b < BINS_ - 1 else 0.0
            s_b = s[b] - s_hi
            c_b = c_lo - c_hi
            nonempty = c_b > 0.0
            total += jnp.where(nonempty, s_b / jnp.where(nonempty, c_b, 1.0), 0.0)
            n += jnp.where(nonempty, 1.0, 0.0)
        out_ref[0] = total / n


def kernel(x, y):
    out = pl.pallas_call(
        _body,
        grid=(GRID,),
        in_specs=[
            pl.BlockSpec((BLK, COLS), lambda i: (i, 0)),
            pl.BlockSpec((BLK, COLS), lambda i: (i, 0)),
        ],
        out_specs=pl.BlockSpec(memory_space=pltpu.SMEM),
        out_shape=jax.ShapeDtypeStruct((1,), jnp.float32),
        scratch_shapes=[pltpu.VMEM((NACC, 8, 128), jnp.float32)],
    )(x, y)
    return out[0]


# speedup vs baseline: 34.3660x; 1.0115x over previous
import jax
import jax.numpy as jnp
from jax.experimental import pallas as pl
from jax.experimental.pallas import tpu as pltpu

BINS_ = 10
ROWS = 4096
COLS = 4096
BLK = 128
GRID = ROWS // BLK
TOTAL = float(ROWS * COLS)
NACC = 2 * BINS_ - 1


def _body(x_ref, y_ref, out_ref, acc_ref):
    pid = pl.program_id(0)

    @pl.when(pid == 0)
    def _init():
        acc_ref[...] = jnp.zeros_like(acc_ref)

    zero = jnp.zeros((8, 128), jnp.float32)
    for r in range(BLK // 8):
        sge = [zero] * BINS_
        cge = [zero] * (BINS_ - 1)
        for c in range(COLS // 128):
            x = x_ref[r * 8:(r + 1) * 8, c * 128:(c + 1) * 128]
            y = y_ref[r * 8:(r + 1) * 8, c * 128:(c + 1) * 128]
            e = jnp.exp(-jnp.abs(x))
            u = 1.0 + e
            s0 = 1.0 / u
            sig = jnp.where(x >= 0.0, s0, 1.0 - s0)
            g = jnp.abs(sig - y)
            pe = jnp.maximum(x, 0.0) - x * y + jnp.log(u)
            sge[0] = sge[0] + pe
            for b in range(1, BINS_):
                mf = jnp.where(g >= float(b) / float(BINS_), 1.0, 0.0)
                sge[b] = sge[b] + pe * mf
                cge[b - 1] = cge[b - 1] + mf

        for b in range(BINS_):
            acc_ref[b] += sge[b]
        for b in range(BINS_ - 1):
            acc_ref[BINS_ + b] += cge[b]

    @pl.when(pid == GRID - 1)
    def _finish():
        s = [jnp.sum(acc_ref[b]) for b in range(BINS_)]
        cg = [jnp.sum(acc_ref[BINS_ + b]) for b in range(BINS_ - 1)]
        total = 0.0
        n = 0.0
        for b in range(BINS_):
            s_hi = s[b + 1] if b + 1 < BINS_ else 0.0
            c_lo = cg[b - 1] if b >= 1 else TOTAL
            c_hi = cg[b] if b < BINS_ - 1 else 0.0
            s_b = s[b] - s_hi
            c_b = c_lo - c_hi
            nonempty = c_b > 0.0
            total += jnp.where(nonempty, s_b / jnp.where(nonempty, c_b, 1.0), 0.0)
            n += jnp.where(nonempty, 1.0, 0.0)
        out_ref[0] = total / n


def kernel(x, y):
    out = pl.pallas_call(
        _body,
        grid=(GRID,),
        in_specs=[
            pl.BlockSpec((BLK, COLS), lambda i: (i, 0)),
            pl.BlockSpec((BLK, COLS), lambda i: (i, 0)),
        ],
        out_specs=pl.BlockSpec(memory_space=pltpu.SMEM),
        out_shape=jax.ShapeDtypeStruct((1,), jnp.float32),
        scratch_shapes=[pltpu.VMEM((NACC, 8, 128), jnp.float32)],
    )(x, y)
    return out[0]
